# Initial kernel scaffold; baseline (speedup 1.0000x reference)
import jax
import jax.numpy as jnp
from jax.experimental import pallas as pl


def kernel(x, sample1_adj, sample2_adj, W_in, b_in, W_convs, b_convs, gammas, betas, W_out, b_out):
    raise NotImplementedError("write your pallas kernel here")


# speedup vs baseline: 5.9632x; 5.9632x over previous
import functools

import jax
import jax.numpy as jnp
from jax import lax
from jax.experimental import pallas as pl
from jax.experimental.pallas import tpu as pltpu
from jax.experimental.pallas import tpu_sc as plsc

N = 10000
D = 256
H = 256
C = 112
L = 8
E = 160000

NC = 2
NS = 16
HH = H // 2

CHUNK = 128
CHUNKS_PER_TILE = 79
EPT = CHUNK * CHUNKS_PER_TILE
E_PAD = EPT * NS

NP = 10240
RPT = NP // NS
ICH = 160
DUMMY = N

RB = 1280
GRID = NP // RB

_mesh = plsc.VectorSubcoreMesh(core_axis_name="c", subcore_axis_name="s", num_cores=NC)


@functools.partial(
    pl.kernel,
    out_type=jax.ShapeDtypeStruct((2, NP), jnp.float32),
    mesh=_mesh,
    scratch_types=[
        pltpu.VMEM((CHUNK,), jnp.int32),
        pltpu.VMEM((CHUNK,), jnp.float32),
        pltpu.VMEM((RPT,), jnp.float32),
        pltpu.VMEM_SHARED((NP,), jnp.float32),
    ],
)
def _deg_kernel(dstb_hbm, deg_hbm, didx, ones, stage, hist):
    c = lax.axis_index("c")
    s = lax.axis_index("s")
    for j in range(CHUNK // 16):
        ones[pl.ds(j * 16, 16)] = jnp.full((16,), 1.0, jnp.float32)

    def zero_body(j, _):
        stage[pl.ds(j * 16, 16)] = jnp.zeros((16,), jnp.float32)
        return 0

    lax.fori_loop(0, RPT // 16, zero_body, 0)
    pltpu.sync_copy(stage, hist.at[pl.ds(s * RPT, RPT)])
    plsc.subcore_barrier()

    ebase = s * EPT

    def edge_body(k, _):
        pltpu.sync_copy(dstb_hbm.at[c, pl.ds(ebase + k * CHUNK, CHUNK)], didx)
        pltpu.sync_copy(ones, hist.at[didx], add=True)
        return 0

    lax.fori_loop(0, CHUNKS_PER_TILE, edge_body, 0)
    plsc.subcore_barrier()
    pltpu.sync_copy(hist.at[pl.ds(s * RPT, RPT)], stage)
    pltpu.sync_copy(stage, deg_hbm.at[c, pl.ds(s * RPT, RPT)])


@functools.partial(
    pl.kernel,
    out_type=jax.ShapeDtypeStruct((2 * NP, HH), jnp.float32),
    mesh=_mesh,
    scratch_types=[
        pltpu.VMEM((CHUNK,), jnp.int32),
        pltpu.VMEM((CHUNK,), jnp.int32),
        pltpu.VMEM((CHUNK, HH), jnp.float32),
        pltpu.VMEM((ICH, HH), jnp.float32),
        pltpu.VMEM_SHARED((NP, HH), jnp.float32),
        pltpu.SemaphoreType.DMA,
    ],
)
def _prop_kernel(m_hbm, srcb_hbm, dstp_hbm, out_hbm, sidx, didx, rows, stage, acc, sem):
    c = lax.axis_index("c")
    s = lax.axis_index("s")
    hbase = c * NP + s * RPT

    def init_body(j, _):
        pltpu.sync_copy(m_hbm.at[pl.ds(hbase + j * ICH, ICH)], stage)
        pltpu.sync_copy(stage, acc.at[pl.ds(s * RPT + j * ICH, ICH)])
        return 0

    lax.fori_loop(0, RPT // ICH, init_body, 0)
    plsc.subcore_barrier()

    ebase = s * EPT

    def edge_body(k, _):
        off = ebase + k * CHUNK
        pltpu.sync_copy(srcb_hbm.at[c, pl.ds(off, CHUNK)], sidx)
        pltpu.sync_copy(dstp_hbm.at[pl.ds(off, CHUNK)], didx)
        pltpu.async_copy(m_hbm.at[sidx], rows, sem).wait()
        pltpu.sync_copy(rows, acc.at[didx], add=True)
        return 0

    lax.fori_loop(0, CHUNKS_PER_TILE, edge_body, 0)
    plsc.subcore_barrier()

    def out_body(j, _):
        pltpu.sync_copy(acc.at[pl.ds(s * RPT + j * ICH, ICH)], stage)
        pltpu.sync_copy(stage, out_hbm.at[pl.ds(hbase + j * ICH, ICH)])
        return 0

    lax.fori_loop(0, RPT // ICH, out_body, 0)


_EPS_SCALE = float(1.0 / (1.0 + 1e-5) ** 0.5)


def _affine_relu(hb0, hb1, degp, bc, sc, be):
    dinv = lax.rsqrt(degp[0, 0, :] + 1.0)[:, None]
    a0 = jnp.maximum((hb0 * dinv + bc[0, 0, :HH]) * sc[0, 0, :HH] + be[0, 0, :HH], 0.0)
    a1 = jnp.maximum((hb1 * dinv + bc[0, 0, HH:]) * sc[0, 0, HH:] + be[0, 0, HH:], 0.0)
    return a0, a1


def _mm_first(x, W_in, b_in):
    def body(x_ref, w_ref, b_ref, out_ref):
        res = jnp.dot(x_ref[...], w_ref[...], preferred_element_type=jnp.float32)
        res = res + b_ref[0, :]
        out_ref[0] = res[:, :HH]
        out_ref[1] = res[:, HH:]

    return pl.pallas_call(
        body,
        grid=(GRID,),
        in_specs=[
            pl.BlockSpec((RB, D), lambda r: (r, 0)),
            pl.BlockSpec((D, H), lambda r: (0, 0)),
            pl.BlockSpec((1, H), lambda r: (0, 0)),
        ],
        out_specs=pl.BlockSpec((2, RB, HH), lambda r: (0, r, 0)),
        out_shape=jax.ShapeDtypeStruct((2, NP, HH), jnp.float32),
    )(x, W_in, b_in.reshape(1, H))


def _mm_conv(h_halves, W_convs, deg, i, a, with_pro, b_convs, scales, betas):
    ap = 0 if i < 1 + L // 2 else 1

    if with_pro:
        def body(h_ref, w_ref, degp_ref, degn_ref, bc_ref, sc_ref, be_ref, out_ref):
            a0, a1 = _affine_relu(h_ref[0], h_ref[1], degp_ref, bc_ref, sc_ref, be_ref)
            res = jnp.dot(a0, w_ref[0, :HH, :], preferred_element_type=jnp.float32)
            res = res + jnp.dot(a1, w_ref[0, HH:, :], preferred_element_type=jnp.float32)
            res = res * lax.rsqrt(degn_ref[0, 0, :] + 1.0)[:, None]
            out_ref[0] = res[:, :HH]
            out_ref[1] = res[:, HH:]

        in_specs = [
            pl.BlockSpec((2, RB, HH), lambda r: (0, r, 0)),
            pl.BlockSpec((1, H, H), lambda r, _i=i: (_i, 0, 0)),
            pl.BlockSpec((1, 1, RB), lambda r, _a=ap: (_a, 0, r)),
            pl.BlockSpec((1, 1, RB), lambda r, _a=a: (_a, 0, r)),
            pl.BlockSpec((1, 1, H), lambda r, _i=i - 1: (_i, 0, 0)),
            pl.BlockSpec((1, 1, H), lambda r, _i=i - 1: (_i, 0, 0)),
            pl.BlockSpec((1, 1, H), lambda r, _i=i - 1: (_i, 0, 0)),
        ]
        args = (h_halves, W_convs, deg.reshape(2, 1, NP), deg.reshape(2, 1, NP),
                b_convs.reshape(L, 1, H), scales.reshape(L, 1, H),
                betas.reshape(L, 1, H))
    else:
        def body(h_ref, w_ref, degn_ref, out_ref):
            res = jnp.dot(h_ref[0], w_ref[0, :HH, :], preferred_element_type=jnp.float32)
            res = res + jnp.dot(h_ref[1], w_ref[0, HH:, :], preferred_element_type=jnp.float32)
            res = res * lax.rsqrt(degn_ref[0, 0, :] + 1.0)[:, None]
            out_ref[0] = res[:, :HH]
            out_ref[1] = res[:, HH:]

        in_specs = [
            pl.BlockSpec((2, RB, HH), lambda r: (0, r, 0)),
            pl.BlockSpec((1, H, H), lambda r, _i=i: (_i, 0, 0)),
            pl.BlockSpec((1, 1, RB), lambda r, _a=a: (_a, 0, r)),
        ]
        args = (h_halves, W_convs, deg.reshape(2, 1, NP))

    return pl.pallas_call(
        body,
        grid=(GRID,),
        in_specs=in_specs,
        out_specs=pl.BlockSpec((2, RB, HH), lambda r: (0, r, 0)),
        out_shape=jax.ShapeDtypeStruct((2, NP, HH), jnp.float32),
    )(*args)


def _mm_final(h_halves, W_out, b_out, deg, b_convs, scales, betas):
    def body(h_ref, w_ref, b_ref, degp_ref, bc_ref, sc_ref, be_ref, out_ref):
        a0, a1 = _affine_relu(h_ref[0], h_ref[1], degp_ref, bc_ref, sc_ref, be_ref)
        res = jnp.dot(a0, w_ref[:HH, :], preferred_element_type=jnp.float32)
        res = res + jnp.dot(a1, w_ref[HH:, :], preferred_element_type=jnp.float32)
        res = res + b_ref[0, :]
        mx = jnp.max(res, axis=1, keepdims=True)
        sh = res - mx
        lse = jnp.log(jnp.sum(jnp.exp(sh), axis=1, keepdims=True))
        out_ref[...] = sh - lse

    return pl.pallas_call(
        body,
        grid=(GRID,),
        in_specs=[
            pl.BlockSpec((2, RB, HH), lambda r: (0, r, 0)),
            pl.BlockSpec((H, C), lambda r: (0, 0)),
            pl.BlockSpec((1, C), lambda r: (0, 0)),
            pl.BlockSpec((1, 1, RB), lambda r: (1, 0, r)),
            pl.BlockSpec((1, 1, H), lambda r: (L - 1, 0, 0)),
            pl.BlockSpec((1, 1, H), lambda r: (L - 1, 0, 0)),
            pl.BlockSpec((1, 1, H), lambda r: (L - 1, 0, 0)),
        ],
        out_specs=pl.BlockSpec((RB, C), lambda r: (r, 0)),
        out_shape=jax.ShapeDtypeStruct((N, C), jnp.float32),
    )(h_halves, W_out, b_out.reshape(1, C), deg.reshape(2, 1, NP),
      b_convs.reshape(L, 1, H), scales.reshape(L, 1, H), betas.reshape(L, 1, H))


def kernel(x, sample1_adj, sample2_adj, W_in, b_in, W_convs, b_convs, gammas,
           betas, W_out, b_out):
    pad_e = E_PAD - E
    srcs, dsts, srcbs = [], [], []
    for adj in (sample1_adj, sample2_adj):
        src = jnp.concatenate([adj[0], jnp.zeros((pad_e,), jnp.int32)])
        dst = jnp.concatenate([adj[1], jnp.full((pad_e,), DUMMY, jnp.int32)])
        srcs.append(src)
        dsts.append(dst)
        srcbs.append(jnp.stack([src, src + NP]))
    dstb = jnp.stack(dsts)

    scales = gammas * _EPS_SCALE

    deg = _deg_kernel(dstb)

    h = _mm_first(x, W_in, b_in)
    for i in range(L):
        a = 0 if i < L // 2 else 1
        m = _mm_conv(h, W_convs, deg, i, a, i > 0, b_convs, scales, betas)
        acc = _prop_kernel(m.reshape(2 * NP, HH), srcbs[a], dstb[a])
        h = acc.reshape(2, NP, HH)

    return _mm_final(h, W_out, b_out, deg, b_convs, scales, betas)


# speedup vs baseline: 8.8515x; 1.4844x over previous
import functools

import jax
import jax.numpy as jnp
from jax import lax
from jax.experimental import pallas as pl
from jax.experimental.pallas import tpu as pltpu
from jax.experimental.pallas import tpu_sc as plsc

N = 10000
D = 256
H = 256
C = 112
L = 8
E = 160000

NC = 2
NS = 16
HH = H // 2

CHUNK = 128
CHUNKS_PER_TILE = 79
EPT = CHUNK * CHUNKS_PER_TILE
E_PAD = EPT * NS

NP = 10240
RPT = NP // NS
ICH = 160
DUMMY = N

RB = 1280
GRID = NP // RB

_mesh = plsc.VectorSubcoreMesh(core_axis_name="c", subcore_axis_name="s", num_cores=NC)


@functools.partial(
    pl.kernel,
    out_type=jax.ShapeDtypeStruct((2, NP), jnp.float32),
    mesh=_mesh,
    scratch_types=[
        pltpu.VMEM((CHUNK,), jnp.int32),
        pltpu.VMEM((CHUNK,), jnp.float32),
        pltpu.VMEM((RPT,), jnp.float32),
        pltpu.VMEM_SHARED((NP,), jnp.float32),
    ],
)
def _deg_kernel(dstb_hbm, deg_hbm, didx, ones, stage, hist):
    c = lax.axis_index("c")
    s = lax.axis_index("s")
    for j in range(CHUNK // 16):
        ones[pl.ds(j * 16, 16)] = jnp.full((16,), 1.0, jnp.float32)

    def zero_body(j, _):
        stage[pl.ds(j * 16, 16)] = jnp.zeros((16,), jnp.float32)
        return 0

    lax.fori_loop(0, RPT // 16, zero_body, 0)
    pltpu.sync_copy(stage, hist.at[pl.ds(s * RPT, RPT)])
    plsc.subcore_barrier()

    ebase = s * EPT

    def edge_body(k, _):
        pltpu.sync_copy(dstb_hbm.at[c, pl.ds(ebase + k * CHUNK, CHUNK)], didx)
        pltpu.sync_copy(ones, hist.at[didx], add=True)
        return 0

    lax.fori_loop(0, CHUNKS_PER_TILE, edge_body, 0)
    plsc.subcore_barrier()
    pltpu.sync_copy(hist.at[pl.ds(s * RPT, RPT)], stage)
    pltpu.sync_copy(stage, deg_hbm.at[c, pl.ds(s * RPT, RPT)])


@functools.partial(
    pl.kernel,
    out_type=jax.ShapeDtypeStruct((2 * NP, HH), jnp.float32),
    mesh=_mesh,
    scratch_types=[
        pltpu.VMEM((CHUNK,), jnp.int32),
        pltpu.VMEM((CHUNK,), jnp.int32),
        pltpu.VMEM((CHUNK,), jnp.int32),
        pltpu.VMEM((CHUNK,), jnp.int32),
        pltpu.VMEM((CHUNK, HH), jnp.float32),
        pltpu.VMEM((CHUNK, HH), jnp.float32),
        pltpu.VMEM_SHARED((NP, HH), jnp.float32),
        pltpu.SemaphoreType.DMA,
        pltpu.SemaphoreType.DMA,
    ],
)
def _prop_kernel(m_hbm, srcb_hbm, dstp_hbm, out_hbm, sidx0, sidx1, didx0, didx1,
                 rows0, rows1, acc, sem0, sem1):
    c = lax.axis_index("c")
    s = lax.axis_index("s")
    hbase = c * NP + s * RPT
    sidx = (sidx0, sidx1)
    didx = (didx0, didx1)
    rows = (rows0, rows1)
    sems = (sem0, sem1)

    pltpu.sync_copy(m_hbm.at[pl.ds(hbase, RPT)], acc.at[pl.ds(s * RPT, RPT)])
    plsc.subcore_barrier()

    pltpu.sync_copy(srcb_hbm.at[c, s, 0], sidx0)
    pltpu.sync_copy(dstp_hbm.at[s, 0], didx0)
    pltpu.async_copy(m_hbm.at[sidx0], rows0, sem0)

    def pair_body(i, _):
        for b in range(2):
            cur = 2 * i + b
            nb = 1 - b
            pltpu.sync_copy(srcb_hbm.at[c, s, cur + 1], sidx[nb])
            pltpu.async_copy(m_hbm.at[sidx[nb]], rows[nb], sems[nb])
            pltpu.sync_copy(dstp_hbm.at[s, cur + 1], didx[nb])
            pltpu.make_async_copy(m_hbm.at[sidx[b]], rows[b], sems[b]).wait()
            pltpu.sync_copy(rows[b], acc.at[didx[b]], add=True)
        return 0

    lax.fori_loop(0, (CHUNKS_PER_TILE - 1) // 2, pair_body, 0)
    lb = (CHUNKS_PER_TILE - 1) % 2
    pltpu.make_async_copy(m_hbm.at[sidx[lb]], rows[lb], sems[lb]).wait()
    pltpu.sync_copy(rows[lb], acc.at[didx[lb]], add=True)
    plsc.subcore_barrier()

    pltpu.sync_copy(acc.at[pl.ds(s * RPT, RPT)], out_hbm.at[pl.ds(hbase, RPT)])


_EPS_SCALE = float(1.0 / (1.0 + 1e-5) ** 0.5)


def _affine_relu(hb0, hb1, degp, bc, sc, be):
    dinv = lax.rsqrt(degp[0, 0, :] + 1.0)[:, None]
    a0 = jnp.maximum((hb0 * dinv + bc[0, 0, :HH]) * sc[0, 0, :HH] + be[0, 0, :HH], 0.0)
    a1 = jnp.maximum((hb1 * dinv + bc[0, 0, HH:]) * sc[0, 0, HH:] + be[0, 0, HH:], 0.0)
    return a0, a1


def _mm_first(x, W_in, b_in):
    def body(x_ref, w_ref, b_ref, out_ref):
        res = jnp.dot(x_ref[...], w_ref[...], preferred_element_type=jnp.float32)
        res = res + b_ref[0, :]
        out_ref[0] = res[:, :HH]
        out_ref[1] = res[:, HH:]

    return pl.pallas_call(
        body,
        grid=(GRID,),
        in_specs=[
            pl.BlockSpec((RB, D), lambda r: (r, 0)),
            pl.BlockSpec((D, H), lambda r: (0, 0)),
            pl.BlockSpec((1, H), lambda r: (0, 0)),
        ],
        out_specs=pl.BlockSpec((2, RB, HH), lambda r: (0, r, 0)),
        out_shape=jax.ShapeDtypeStruct((2, NP, HH), jnp.float32),
    )(x, W_in, b_in.reshape(1, H))


def _mm_conv(h_halves, W_convs, deg, i, a, with_pro, b_convs, scales, betas):
    ap = 0 if i < 1 + L // 2 else 1

    if with_pro:
        def body(h_ref, w_ref, degp_ref, degn_ref, bc_ref, sc_ref, be_ref, out_ref):
            a0, a1 = _affine_relu(h_ref[0], h_ref[1], degp_ref, bc_ref, sc_ref, be_ref)
            res = jnp.dot(a0, w_ref[0, :HH, :], preferred_element_type=jnp.float32)
            res = res + jnp.dot(a1, w_ref[0, HH:, :], preferred_element_type=jnp.float32)
            res = res * lax.rsqrt(degn_ref[0, 0, :] + 1.0)[:, None]
            out_ref[0] = res[:, :HH]
            out_ref[1] = res[:, HH:]

        in_specs = [
            pl.BlockSpec((2, RB, HH), lambda r: (0, r, 0)),
            pl.BlockSpec((1, H, H), lambda r, _i=i: (_i, 0, 0)),
            pl.BlockSpec((1, 1, RB), lambda r, _a=ap: (_a, 0, r)),
            pl.BlockSpec((1, 1, RB), lambda r, _a=a: (_a, 0, r)),
            pl.BlockSpec((1, 1, H), lambda r, _i=i - 1: (_i, 0, 0)),
            pl.BlockSpec((1, 1, H), lambda r, _i=i - 1: (_i, 0, 0)),
            pl.BlockSpec((1, 1, H), lambda r, _i=i - 1: (_i, 0, 0)),
        ]
        args = (h_halves, W_convs, deg.reshape(2, 1, NP), deg.reshape(2, 1, NP),
                b_convs.reshape(L, 1, H), scales.reshape(L, 1, H),
                betas.reshape(L, 1, H))
    else:
        def body(h_ref, w_ref, degn_ref, out_ref):
            res = jnp.dot(h_ref[0], w_ref[0, :HH, :], preferred_element_type=jnp.float32)
            res = res + jnp.dot(h_ref[1], w_ref[0, HH:, :], preferred_element_type=jnp.float32)
            res = res * lax.rsqrt(degn_ref[0, 0, :] + 1.0)[:, None]
            out_ref[0] = res[:, :HH]
            out_ref[1] = res[:, HH:]

        in_specs = [
            pl.BlockSpec((2, RB, HH), lambda r: (0, r, 0)),
            pl.BlockSpec((1, H, H), lambda r, _i=i: (_i, 0, 0)),
            pl.BlockSpec((1, 1, RB), lambda r, _a=a: (_a, 0, r)),
        ]
        args = (h_halves, W_convs, deg.reshape(2, 1, NP))

    return pl.pallas_call(
        body,
        grid=(GRID,),
        in_specs=in_specs,
        out_specs=pl.BlockSpec((2, RB, HH), lambda r: (0, r, 0)),
        out_shape=jax.ShapeDtypeStruct((2, NP, HH), jnp.float32),
    )(*args)


def _mm_final(h_halves, W_out, b_out, deg, b_convs, scales, betas):
    def body(h_ref, w_ref, b_ref, degp_ref, bc_ref, sc_ref, be_ref, out_ref):
        a0, a1 = _affine_relu(h_ref[0], h_ref[1], degp_ref, bc_ref, sc_ref, be_ref)
        res = jnp.dot(a0, w_ref[:HH, :], preferred_element_type=jnp.float32)
        res = res + jnp.dot(a1, w_ref[HH:, :], preferred_element_type=jnp.float32)
        res = res + b_ref[0, :]
        mx = jnp.max(res, axis=1, keepdims=True)
        sh = res - mx
        lse = jnp.log(jnp.sum(jnp.exp(sh), axis=1, keepdims=True))
        out_ref[...] = sh - lse

    return pl.pallas_call(
        body,
        grid=(GRID,),
        in_specs=[
            pl.BlockSpec((2, RB, HH), lambda r: (0, r, 0)),
            pl.BlockSpec((H, C), lambda r: (0, 0)),
            pl.BlockSpec((1, C), lambda r: (0, 0)),
            pl.BlockSpec((1, 1, RB), lambda r: (1, 0, r)),
            pl.BlockSpec((1, 1, H), lambda r: (L - 1, 0, 0)),
            pl.BlockSpec((1, 1, H), lambda r: (L - 1, 0, 0)),
            pl.BlockSpec((1, 1, H), lambda r: (L - 1, 0, 0)),
        ],
        out_specs=pl.BlockSpec((RB, C), lambda r: (r, 0)),
        out_shape=jax.ShapeDtypeStruct((N, C), jnp.float32),
    )(h_halves, W_out, b_out.reshape(1, C), deg.reshape(2, 1, NP),
      b_convs.reshape(L, 1, H), scales.reshape(L, 1, H), betas.reshape(L, 1, H))


def kernel(x, sample1_adj, sample2_adj, W_in, b_in, W_convs, b_convs, gammas,
           betas, W_out, b_out):
    pad_e = E_PAD - E
    srcs, dsts, srcbs = [], [], []
    for adj in (sample1_adj, sample2_adj):
        src = jnp.concatenate([adj[0], jnp.zeros((pad_e,), jnp.int32)])
        dst = jnp.concatenate([adj[1], jnp.full((pad_e,), DUMMY, jnp.int32)])
        srcs.append(src)
        dsts.append(dst)
        srcbs.append(jnp.stack([src, src + NP]))
    dstb = jnp.stack(dsts)

    scales = gammas * _EPS_SCALE

    deg = _deg_kernel(dstb)

    h = _mm_first(x, W_in, b_in)
    for i in range(L):
        a = 0 if i < L // 2 else 1
        m = _mm_conv(h, W_convs, deg, i, a, i > 0, b_convs, scales, betas)
        acc = _prop_kernel(m.reshape(2 * NP, HH),
                           srcbs[a].reshape(2, NS, CHUNKS_PER_TILE, CHUNK),
                           dstb[a].reshape(NS, CHUNKS_PER_TILE, CHUNK))
        h = acc.reshape(2, NP, HH)

    return _mm_final(h, W_out, b_out, deg, b_convs, scales, betas)


# speedup vs baseline: 9.7709x; 1.1039x over previous
import functools

import jax
import jax.numpy as jnp
from jax import lax
from jax.experimental import pallas as pl
from jax.experimental.pallas import tpu as pltpu
from jax.experimental.pallas import tpu_sc as plsc

N = 10000
D = 256
H = 256
C = 112
L = 8
E = 160000

NC = 2
NS = 16
HH = H // 2

CHUNK = 128
CHUNKS_PER_TILE = 79
EPT = CHUNK * CHUNKS_PER_TILE
E_PAD = EPT * NS

NP = 10240
RPT = NP // NS
ICH = 160
DUMMY = N

RB = 1280
GRID = NP // RB

_mesh = plsc.VectorSubcoreMesh(core_axis_name="c", subcore_axis_name="s", num_cores=NC)


@functools.partial(
    pl.kernel,
    out_type=jax.ShapeDtypeStruct((2, NP), jnp.float32),
    mesh=_mesh,
    scratch_types=[
        pltpu.VMEM((CHUNK,), jnp.int32),
        pltpu.VMEM((CHUNK,), jnp.float32),
        pltpu.VMEM((RPT,), jnp.float32),
        pltpu.VMEM_SHARED((NP,), jnp.float32),
    ],
)
def _deg_kernel(dstb_hbm, deg_hbm, didx, ones, stage, hist):
    c = lax.axis_index("c")
    s = lax.axis_index("s")
    for j in range(CHUNK // 16):
        ones[pl.ds(j * 16, 16)] = jnp.full((16,), 1.0, jnp.float32)

    def zero_body(j, _):
        stage[pl.ds(j * 16, 16)] = jnp.zeros((16,), jnp.float32)
        return 0

    lax.fori_loop(0, RPT // 16, zero_body, 0)
    pltpu.sync_copy(stage, hist.at[pl.ds(s * RPT, RPT)])
    plsc.subcore_barrier()

    ebase = s * EPT

    def edge_body(k, _):
        pltpu.sync_copy(dstb_hbm.at[c, pl.ds(ebase + k * CHUNK, CHUNK)], didx)
        pltpu.sync_copy(ones, hist.at[didx], add=True)
        return 0

    lax.fori_loop(0, CHUNKS_PER_TILE, edge_body, 0)
    plsc.subcore_barrier()
    pltpu.sync_copy(hist.at[pl.ds(s * RPT, RPT)], stage)
    pltpu.sync_copy(stage, deg_hbm.at[c, pl.ds(s * RPT, RPT)])


@functools.partial(
    pl.kernel,
    out_type=jax.ShapeDtypeStruct((2 * NP, HH), jnp.float32),
    mesh=_mesh,
    scratch_types=[
        pltpu.VMEM((2, CHUNK), jnp.int32),
        pltpu.VMEM((2, CHUNK), jnp.int32),
        pltpu.VMEM((CHUNK, HH), jnp.float32),
        pltpu.VMEM((CHUNK, HH), jnp.float32),
        pltpu.VMEM_SHARED((NP, HH), jnp.float32),
        pltpu.SemaphoreType.DMA,
        pltpu.SemaphoreType.DMA,
    ],
)
def _prop_kernel(m_hbm, sd_hbm, out_hbm, sd0, sd1, rows0, rows1, acc, sem0, sem1):
    c = lax.axis_index("c")
    s = lax.axis_index("s")
    hbase = c * NP + s * RPT
    sd = (sd0, sd1)
    rows = (rows0, rows1)
    sems = (sem0, sem1)

    def fetch(k, b):
        pltpu.sync_copy(sd_hbm.at[c, s, k], sd[b])

    def gather(b):
        pltpu.async_copy(m_hbm.at[sd[b].at[0]], rows[b], sems[b])

    def wait_gather(b):
        pltpu.make_async_copy(m_hbm.at[sd[b].at[0]], rows[b], sems[b]).wait()

    def scatter(b):
        pltpu.sync_copy(rows[b], acc.at[sd[b].at[1]], add=True)

    fetch(0, 0)
    gather(0)
    fetch(1, 1)
    gather(1)

    pltpu.sync_copy(m_hbm.at[pl.ds(hbase, RPT)], acc.at[pl.ds(s * RPT, RPT)])
    plsc.subcore_barrier()

    wait_gather(0)
    scatter(0)

    def pair_body(i, _):
        for b01 in range(2):
            cur = 2 * i + 1 + b01
            bb = 1 - b01
            fetch(cur + 1, b01)
            gather(b01)
            wait_gather(bb)
            scatter(bb)
        return 0

    lax.fori_loop(0, (CHUNKS_PER_TILE - 3) // 2, pair_body, 0)
    fetch(CHUNKS_PER_TILE - 1, 0)
    gather(0)
    wait_gather(1)
    scatter(1)
    wait_gather(0)
    scatter(0)
    plsc.subcore_barrier()

    pltpu.sync_copy(acc.at[pl.ds(s * RPT, RPT)], out_hbm.at[pl.ds(hbase, RPT)])


_EPS_SCALE = float(1.0 / (1.0 + 1e-5) ** 0.5)


def _affine_relu(hb0, hb1, degp, bc, sc, be):
    dinv = lax.rsqrt(degp[0, 0, :] + 1.0)[:, None]
    a0 = jnp.maximum((hb0 * dinv + bc[0, 0, :HH]) * sc[0, 0, :HH] + be[0, 0, :HH], 0.0)
    a1 = jnp.maximum((hb1 * dinv + bc[0, 0, HH:]) * sc[0, 0, HH:] + be[0, 0, HH:], 0.0)
    return a0, a1


def _mm_first(x, W_in, b_in):
    def body(x_ref, w_ref, b_ref, out_ref):
        res = jnp.dot(x_ref[...], w_ref[...], preferred_element_type=jnp.float32)
        res = res + b_ref[0, :]
        out_ref[0] = res[:, :HH]
        out_ref[1] = res[:, HH:]

    return pl.pallas_call(
        body,
        grid=(GRID,),
        in_specs=[
            pl.BlockSpec((RB, D), lambda r: (r, 0)),
            pl.BlockSpec((D, H), lambda r: (0, 0)),
            pl.BlockSpec((1, H), lambda r: (0, 0)),
        ],
        out_specs=pl.BlockSpec((2, RB, HH), lambda r: (0, r, 0)),
        out_shape=jax.ShapeDtypeStruct((2, NP, HH), jnp.float32),
    )(x, W_in, b_in.reshape(1, H))


def _mm_conv(h_halves, W_convs, deg, i, a, with_pro, b_convs, scales, betas):
    ap = 0 if i < 1 + L // 2 else 1

    if with_pro:
        def body(h_ref, w_ref, degp_ref, degn_ref, bc_ref, sc_ref, be_ref, out_ref):
            a0, a1 = _affine_relu(h_ref[0], h_ref[1], degp_ref, bc_ref, sc_ref, be_ref)
            res = jnp.dot(a0, w_ref[0, :HH, :], preferred_element_type=jnp.float32)
            res = res + jnp.dot(a1, w_ref[0, HH:, :], preferred_element_type=jnp.float32)
            res = res * lax.rsqrt(degn_ref[0, 0, :] + 1.0)[:, None]
            out_ref[0] = res[:, :HH]
            out_ref[1] = res[:, HH:]

        in_specs = [
            pl.BlockSpec((2, RB, HH), lambda r: (0, r, 0)),
            pl.BlockSpec((1, H, H), lambda r, _i=i: (_i, 0, 0)),
            pl.BlockSpec((1, 1, RB), lambda r, _a=ap: (_a, 0, r)),
            pl.BlockSpec((1, 1, RB), lambda r, _a=a: (_a, 0, r)),
            pl.BlockSpec((1, 1, H), lambda r, _i=i - 1: (_i, 0, 0)),
            pl.BlockSpec((1, 1, H), lambda r, _i=i - 1: (_i, 0, 0)),
            pl.BlockSpec((1, 1, H), lambda r, _i=i - 1: (_i, 0, 0)),
        ]
        args = (h_halves, W_convs, deg.reshape(2, 1, NP), deg.reshape(2, 1, NP),
                b_convs.reshape(L, 1, H), scales.reshape(L, 1, H),
                betas.reshape(L, 1, H))
    else:
        def body(h_ref, w_ref, degn_ref, out_ref):
            res = jnp.dot(h_ref[0], w_ref[0, :HH, :], preferred_element_type=jnp.float32)
            res = res + jnp.dot(h_ref[1], w_ref[0, HH:, :], preferred_element_type=jnp.float32)
            res = res * lax.rsqrt(degn_ref[0, 0, :] + 1.0)[:, None]
            out_ref[0] = res[:, :HH]
            out_ref[1] = res[:, HH:]

        in_specs = [
            pl.BlockSpec((2, RB, HH), lambda r: (0, r, 0)),
            pl.BlockSpec((1, H, H), lambda r, _i=i: (_i, 0, 0)),
            pl.BlockSpec((1, 1, RB), lambda r, _a=a: (_a, 0, r)),
        ]
        args = (h_halves, W_convs, deg.reshape(2, 1, NP))

    return pl.pallas_call(
        body,
        grid=(GRID,),
        in_specs=in_specs,
        out_specs=pl.BlockSpec((2, RB, HH), lambda r: (0, r, 0)),
        out_shape=jax.ShapeDtypeStruct((2, NP, HH), jnp.float32),
    )(*args)


def _mm_final(h_halves, W_out, b_out, deg, b_convs, scales, betas):
    def body(h_ref, w_ref, b_ref, degp_ref, bc_ref, sc_ref, be_ref, out_ref):
        a0, a1 = _affine_relu(h_ref[0], h_ref[1], degp_ref, bc_ref, sc_ref, be_ref)
        res = jnp.dot(a0, w_ref[:HH, :], preferred_element_type=jnp.float32)
        res = res + jnp.dot(a1, w_ref[HH:, :], preferred_element_type=jnp.float32)
        res = res + b_ref[0, :]
        mx = jnp.max(res, axis=1, keepdims=True)
        sh = res - mx
        lse = jnp.log(jnp.sum(jnp.exp(sh), axis=1, keepdims=True))
        out_ref[...] = sh - lse

    return pl.pallas_call(
        body,
        grid=(GRID,),
        in_specs=[
            pl.BlockSpec((2, RB, HH), lambda r: (0, r, 0)),
            pl.BlockSpec((H, C), lambda r: (0, 0)),
            pl.BlockSpec((1, C), lambda r: (0, 0)),
            pl.BlockSpec((1, 1, RB), lambda r: (1, 0, r)),
            pl.BlockSpec((1, 1, H), lambda r: (L - 1, 0, 0)),
            pl.BlockSpec((1, 1, H), lambda r: (L - 1, 0, 0)),
            pl.BlockSpec((1, 1, H), lambda r: (L - 1, 0, 0)),
        ],
        out_specs=pl.BlockSpec((RB, C), lambda r: (r, 0)),
        out_shape=jax.ShapeDtypeStruct((N, C), jnp.float32),
    )(h_halves, W_out, b_out.reshape(1, C), deg.reshape(2, 1, NP),
      b_convs.reshape(L, 1, H), scales.reshape(L, 1, H), betas.reshape(L, 1, H))


def kernel(x, sample1_adj, sample2_adj, W_in, b_in, W_convs, b_convs, gammas,
           betas, W_out, b_out):
    pad_e = E_PAD - E
    dsts, sds = [], []
    for adj in (sample1_adj, sample2_adj):
        src = jnp.concatenate([adj[0], jnp.zeros((pad_e,), jnp.int32)])
        dst = jnp.concatenate([adj[1], jnp.full((pad_e,), DUMMY, jnp.int32)])
        dsts.append(dst)
        halves = [
            jnp.stack([src + cc * NP, dst])
               .reshape(2, NS, CHUNKS_PER_TILE, CHUNK)
               .transpose(1, 2, 0, 3)
            for cc in range(2)
        ]
        sds.append(jnp.stack(halves))
    dstb = jnp.stack(dsts)

    scales = gammas * _EPS_SCALE

    deg = _deg_kernel(dstb)

    h = _mm_first(x, W_in, b_in)
    for i in range(L):
        a = 0 if i < L // 2 else 1
        m = _mm_conv(h, W_convs, deg, i, a, i > 0, b_convs, scales, betas)
        acc = _prop_kernel(m.reshape(2 * NP, HH), sds[a])
        h = acc.reshape(2, NP, HH)

    return _mm_final(h, W_out, b_out, deg, b_convs, scales, betas)


# speedup vs baseline: 10.4056x; 1.0650x over previous
import functools

import jax
import jax.numpy as jnp
from jax import lax
from jax.experimental import pallas as pl
from jax.experimental.pallas import tpu as pltpu
from jax.experimental.pallas import tpu_sc as plsc

N = 10000
D = 256
H = 256
C = 112
L = 8
E = 160000

NC = 2
NS = 16
HH = H // 2

CHUNK = 128
CHUNKS_PER_TILE = 79
EPT = CHUNK * CHUNKS_PER_TILE
E_PAD = EPT * NS

NP = 10240
RPT = NP // NS
ICH = 160
DUMMY = N

RB = 1280
GRID = NP // RB

_mesh = plsc.VectorSubcoreMesh(core_axis_name="c", subcore_axis_name="s", num_cores=NC)


@functools.partial(
    pl.kernel,
    out_type=jax.ShapeDtypeStruct((2, NP), jnp.float32),
    mesh=_mesh,
    scratch_types=[
        pltpu.VMEM((CHUNK,), jnp.int32),
        pltpu.VMEM((CHUNK,), jnp.float32),
        pltpu.VMEM((RPT,), jnp.float32),
        pltpu.VMEM_SHARED((NP,), jnp.float32),
    ],
)
def _deg_kernel(dstb_hbm, deg_hbm, didx, ones, stage, hist):
    c = lax.axis_index("c")
    s = lax.axis_index("s")
    for j in range(CHUNK // 16):
        ones[pl.ds(j * 16, 16)] = jnp.full((16,), 1.0, jnp.float32)

    def zero_body(j, _):
        stage[pl.ds(j * 16, 16)] = jnp.zeros((16,), jnp.float32)
        return 0

    lax.fori_loop(0, RPT // 16, zero_body, 0)
    pltpu.sync_copy(stage, hist.at[pl.ds(s * RPT, RPT)])
    plsc.subcore_barrier()

    ebase = s * EPT

    def edge_body(k, _):
        pltpu.sync_copy(dstb_hbm.at[c, pl.ds(ebase + k * CHUNK, CHUNK)], didx)
        pltpu.sync_copy(ones, hist.at[didx], add=True)
        return 0

    lax.fori_loop(0, CHUNKS_PER_TILE, edge_body, 0)
    plsc.subcore_barrier()
    pltpu.sync_copy(hist.at[pl.ds(s * RPT, RPT)], stage)
    pltpu.sync_copy(stage, deg_hbm.at[c, pl.ds(s * RPT, RPT)])


@functools.partial(
    pl.kernel,
    out_type=jax.ShapeDtypeStruct((2 * NP, HH), jnp.float32),
    mesh=_mesh,
    scratch_types=[
        pltpu.VMEM((2, CHUNK), jnp.int32),
        pltpu.VMEM((2, CHUNK), jnp.int32),
        pltpu.VMEM((2, CHUNK), jnp.int32),
        pltpu.VMEM((CHUNK, HH), jnp.float32),
        pltpu.VMEM((CHUNK, HH), jnp.float32),
        pltpu.VMEM_SHARED((NP, HH), jnp.float32),
        pltpu.SemaphoreType.DMA,
        pltpu.SemaphoreType.DMA,
        pltpu.SemaphoreType.DMA,
        pltpu.SemaphoreType.DMA,
        pltpu.SemaphoreType.DMA,
    ],
)
def _prop_kernel(m_hbm, sd_hbm, out_hbm, sd0, sd1, sd2, rows0, rows1, acc,
                 semg0, semg1, semi0, semi1, semi2):
    c = lax.axis_index("c")
    s = lax.axis_index("s")
    hbase = c * NP + s * RPT
    sd = (sd0, sd1, sd2)
    semi = (semi0, semi1, semi2)
    rows = (rows0, rows1)
    semg = (semg0, semg1)

    def fetch(k, j):
        pltpu.async_copy(sd_hbm.at[c, s, k], sd[j], semi[j])

    def wait_fetch(k, j):
        pltpu.make_async_copy(sd_hbm.at[c, s, k], sd[j], semi[j]).wait()

    def gather(j, b):
        pltpu.async_copy(m_hbm.at[sd[j].at[0]], rows[b], semg[b])

    def wait_gather(j, b):
        pltpu.make_async_copy(m_hbm.at[sd[j].at[0]], rows[b], semg[b]).wait()

    def scatter(j, b):
        pltpu.sync_copy(rows[b], acc.at[sd[j].at[1]], add=True)

    fetch(0, 0)
    fetch(1, 1)
    fetch(2, 2)
    wait_fetch(0, 0)
    gather(0, 0)
    wait_fetch(1, 1)
    gather(1, 1)

    pltpu.sync_copy(m_hbm.at[pl.ds(hbase, RPT)], acc.at[pl.ds(s * RPT, RPT)])
    plsc.subcore_barrier()

    wait_gather(0, 0)
    scatter(0, 0)
    fetch(3, 0)

    def group_body(i, _):
        base = 6 * i
        for t in range(6):
            cur = base + 1 + t
            jc, bc = (1 + t) % 3, (1 + t) % 2
            jn, bn = (2 + t) % 3, (t) % 2
            wait_fetch(cur + 1, jn)
            gather(jn, bn)
            wait_gather(jc, bc)
            scatter(jc, bc)
            fetch(cur + 3, jc)
        return 0

    lax.fori_loop(0, (CHUNKS_PER_TILE - 7) // 6, group_body, 0)
    for t in range(6):
        cur = CHUNKS_PER_TILE - 6 + t
        jc, bc = cur % 3, cur % 2
        if cur + 1 < CHUNKS_PER_TILE:
            jn, bn = (cur + 1) % 3, (cur + 1) % 2
            wait_fetch(cur + 1, jn)
            gather(jn, bn)
        wait_gather(jc, bc)
        scatter(jc, bc)
        if cur + 3 < CHUNKS_PER_TILE:
            fetch(cur + 3, (cur + 3) % 3)
    plsc.subcore_barrier()

    pltpu.sync_copy(acc.at[pl.ds(s * RPT, RPT)], out_hbm.at[pl.ds(hbase, RPT)])


_EPS_SCALE = float(1.0 / (1.0 + 1e-5) ** 0.5)


def _affine_relu(hb0, hb1, degp, bc, sc, be):
    dinv = lax.rsqrt(degp[0, 0, :] + 1.0)[:, None]
    a0 = jnp.maximum((hb0 * dinv + bc[0, 0, :HH]) * sc[0, 0, :HH] + be[0, 0, :HH], 0.0)
    a1 = jnp.maximum((hb1 * dinv + bc[0, 0, HH:]) * sc[0, 0, HH:] + be[0, 0, HH:], 0.0)
    return a0, a1


def _mm_first(x, W_in, b_in):
    def body(x_ref, w_ref, b_ref, out_ref):
        res = jnp.dot(x_ref[...], w_ref[...], preferred_element_type=jnp.float32)
        res = res + b_ref[0, :]
        out_ref[0] = res[:, :HH]
        out_ref[1] = res[:, HH:]

    return pl.pallas_call(
        body,
        grid=(GRID,),
        in_specs=[
            pl.BlockSpec((RB, D), lambda r: (r, 0)),
            pl.BlockSpec((D, H), lambda r: (0, 0)),
            pl.BlockSpec((1, H), lambda r: (0, 0)),
        ],
        out_specs=pl.BlockSpec((2, RB, HH), lambda r: (0, r, 0)),
        out_shape=jax.ShapeDtypeStruct((2, NP, HH), jnp.float32),
    )(x, W_in, b_in.reshape(1, H))


def _mm_conv(h_halves, W_convs, deg, i, a, with_pro, b_convs, scales, betas):
    ap = 0 if i < 1 + L // 2 else 1

    if with_pro:
        def body(h_ref, w_ref, degp_ref, degn_ref, bc_ref, sc_ref, be_ref, out_ref):
            a0, a1 = _affine_relu(h_ref[0], h_ref[1], degp_ref, bc_ref, sc_ref, be_ref)
            res = jnp.dot(a0, w_ref[0, :HH, :], preferred_element_type=jnp.float32)
            res = res + jnp.dot(a1, w_ref[0, HH:, :], preferred_element_type=jnp.float32)
            res = res * lax.rsqrt(degn_ref[0, 0, :] + 1.0)[:, None]
            out_ref[0] = res[:, :HH]
            out_ref[1] = res[:, HH:]

        in_specs = [
            pl.BlockSpec((2, RB, HH), lambda r: (0, r, 0)),
            pl.BlockSpec((1, H, H), lambda r, _i=i: (_i, 0, 0)),
            pl.BlockSpec((1, 1, RB), lambda r, _a=ap: (_a, 0, r)),
            pl.BlockSpec((1, 1, RB), lambda r, _a=a: (_a, 0, r)),
            pl.BlockSpec((1, 1, H), lambda r, _i=i - 1: (_i, 0, 0)),
            pl.BlockSpec((1, 1, H), lambda r, _i=i - 1: (_i, 0, 0)),
            pl.BlockSpec((1, 1, H), lambda r, _i=i - 1: (_i, 0, 0)),
        ]
        args = (h_halves, W_convs, deg.reshape(2, 1, NP), deg.reshape(2, 1, NP),
                b_convs.reshape(L, 1, H), scales.reshape(L, 1, H),
                betas.reshape(L, 1, H))
    else:
        def body(h_ref, w_ref, degn_ref, out_ref):
            res = jnp.dot(h_ref[0], w_ref[0, :HH, :], preferred_element_type=jnp.float32)
            res = res + jnp.dot(h_ref[1], w_ref[0, HH:, :], preferred_element_type=jnp.float32)
            res = res * lax.rsqrt(degn_ref[0, 0, :] + 1.0)[:, None]
            out_ref[0] = res[:, :HH]
            out_ref[1] = res[:, HH:]

        in_specs = [
            pl.BlockSpec((2, RB, HH), lambda r: (0, r, 0)),
            pl.BlockSpec((1, H, H), lambda r, _i=i: (_i, 0, 0)),
            pl.BlockSpec((1, 1, RB), lambda r, _a=a: (_a, 0, r)),
        ]
        args = (h_halves, W_convs, deg.reshape(2, 1, NP))

    return pl.pallas_call(
        body,
        grid=(GRID,),
        in_specs=in_specs,
        out_specs=pl.BlockSpec((2, RB, HH), lambda r: (0, r, 0)),
        out_shape=jax.ShapeDtypeStruct((2, NP, HH), jnp.float32),
    )(*args)


def _mm_final(h_halves, W_out, b_out, deg, b_convs, scales, betas):
    def body(h_ref, w_ref, b_ref, degp_ref, bc_ref, sc_ref, be_ref, out_ref):
        a0, a1 = _affine_relu(h_ref[0], h_ref[1], degp_ref, bc_ref, sc_ref, be_ref)
        res = jnp.dot(a0, w_ref[:HH, :], preferred_element_type=jnp.float32)
        res = res + jnp.dot(a1, w_ref[HH:, :], preferred_element_type=jnp.float32)
        res = res + b_ref[0, :]
        mx = jnp.max(res, axis=1, keepdims=True)
        sh = res - mx
        lse = jnp.log(jnp.sum(jnp.exp(sh), axis=1, keepdims=True))
        out_ref[...] = sh - lse

    return pl.pallas_call(
        body,
        grid=(GRID,),
        in_specs=[
            pl.BlockSpec((2, RB, HH), lambda r: (0, r, 0)),
            pl.BlockSpec((H, C), lambda r: (0, 0)),
            pl.BlockSpec((1, C), lambda r: (0, 0)),
            pl.BlockSpec((1, 1, RB), lambda r: (1, 0, r)),
            pl.BlockSpec((1, 1, H), lambda r: (L - 1, 0, 0)),
            pl.BlockSpec((1, 1, H), lambda r: (L - 1, 0, 0)),
            pl.BlockSpec((1, 1, H), lambda r: (L - 1, 0, 0)),
        ],
        out_specs=pl.BlockSpec((RB, C), lambda r: (r, 0)),
        out_shape=jax.ShapeDtypeStruct((N, C), jnp.float32),
    )(h_halves, W_out, b_out.reshape(1, C), deg.reshape(2, 1, NP),
      b_convs.reshape(L, 1, H), scales.reshape(L, 1, H), betas.reshape(L, 1, H))


def kernel(x, sample1_adj, sample2_adj, W_in, b_in, W_convs, b_convs, gammas,
           betas, W_out, b_out):
    pad_e = E_PAD - E
    dsts, sds = [], []
    for adj in (sample1_adj, sample2_adj):
        src = jnp.concatenate([adj[0], jnp.zeros((pad_e,), jnp.int32)])
        dst = jnp.concatenate([adj[1], jnp.full((pad_e,), DUMMY, jnp.int32)])
        dsts.append(dst)
        halves = [
            jnp.stack([src + cc * NP, dst])
               .reshape(2, NS, CHUNKS_PER_TILE, CHUNK)
               .transpose(1, 2, 0, 3)
            for cc in range(2)
        ]
        sds.append(jnp.stack(halves))
    dstb = jnp.stack(dsts)

    scales = gammas * _EPS_SCALE

    deg = _deg_kernel(dstb)

    h = _mm_first(x, W_in, b_in)
    for i in range(L):
        a = 0 if i < L // 2 else 1
        m = _mm_conv(h, W_convs, deg, i, a, i > 0, b_convs, scales, betas)
        acc = _prop_kernel(m.reshape(2 * NP, HH), sds[a])
        h = acc.reshape(2, NP, HH)

    return _mm_final(h, W_out, b_out, deg, b_convs, scales, betas)
